# Initial kernel scaffold; baseline (speedup 1.0000x reference)
import jax
import jax.numpy as jnp
from jax.experimental import pallas as pl


def kernel(t_users, user_id, t_items, item_id, F_u, E_u, Tw_u, Tb_u, F_i, E_i, Tw_i, Tb_i, final_bias):
    raise NotImplementedError("write your pallas kernel here")


# speedup vs baseline: 2.4472x; 2.4472x over previous
import functools

import jax
import jax.numpy as jnp
from jax import lax
from jax.experimental import pallas as pl
from jax.experimental.pallas import tpu as pltpu
from jax.experimental.pallas import tpu_sc as plsc

_B = 16384
_FV = 64
_NE = 16
_NC = 2
_NS = 16
_NW = _NC * _NS
_BPW = _B // _NW
_CH = 128
_NCH = _BPW // _CH

_BLK = 2048
_NB = _B // _BLK


def _sc_gather_body(uid_hbm, iid_hbm, eu_hbm, ei_hbm, out_u, out_i,
                    uidx_v, iidx_v, urows_v, irows_v, sem):
    wid = lax.axis_index("s") * _NC + lax.axis_index("c")
    base = wid * _BPW
    crow = wid * _NCH
    pltpu.sync_copy(uid_hbm.at[pl.ds(crow, _NCH)], uidx_v)
    pltpu.sync_copy(iid_hbm.at[pl.ds(crow, _NCH)], iidx_v)
    cps = []
    for j in range(_NCH):
        cps.append(pltpu.async_copy(
            eu_hbm.at[uidx_v.at[j]], urows_v.at[pl.ds(j * _CH, _CH)], sem))
        cps.append(pltpu.async_copy(
            ei_hbm.at[iidx_v.at[j]], irows_v.at[pl.ds(j * _CH, _CH)], sem))
    for cp in cps:
        cp.wait()
    pltpu.sync_copy(urows_v, out_u.at[pl.ds(base, _BPW)])
    pltpu.sync_copy(irows_v, out_i.at[pl.ds(base, _BPW)])


_sc_gather = functools.partial(
    pl.kernel,
    mesh=plsc.VectorSubcoreMesh(core_axis_name="c", subcore_axis_name="s"),
    out_type=[jax.ShapeDtypeStruct((_B, _NE), jnp.float32),
              jax.ShapeDtypeStruct((_B, _NE), jnp.float32)],
    scratch_types=[
        pltpu.VMEM((_NCH, _CH), jnp.int32),
        pltpu.VMEM((_NCH, _CH), jnp.int32),
        pltpu.VMEM((_BPW, _NE), jnp.float32),
        pltpu.VMEM((_BPW, _NE), jnp.float32),
        pltpu.SemaphoreType.DMA,
    ],
    compiler_params=pltpu.CompilerParams(use_tc_tiling_on_sc=False),
)(_sc_gather_body)


def _tc_body(c0_ref, tbu_ref, tbi_ref, twu_ref, twi_ref,
             tu_ref, ti_ref, eu_ref, ei_ref, out_ref):
    tu = jnp.sum(tu_ref[...] * twu_ref[...], axis=1) + tbu_ref[0]
    ti = jnp.sum(ti_ref[...] * twi_ref[...], axis=1) + tbi_ref[0]
    ed = jnp.sum(eu_ref[...] * ei_ref[...], axis=1)
    out_ref[0, 0, :] = tu * ti + ed + c0_ref[0]


_tc_combine = pl.pallas_call(
    _tc_body,
    grid=(_NB,),
    in_specs=[
        pl.BlockSpec(memory_space=pltpu.SMEM),
        pl.BlockSpec(memory_space=pltpu.SMEM),
        pl.BlockSpec(memory_space=pltpu.SMEM),
        pl.BlockSpec((1, _FV), lambda i: (0, 0)),
        pl.BlockSpec((1, _FV), lambda i: (0, 0)),
        pl.BlockSpec((_BLK, _FV), lambda i: (i, 0)),
        pl.BlockSpec((_BLK, _FV), lambda i: (i, 0)),
        pl.BlockSpec((_BLK, _NE), lambda i: (i, 0)),
        pl.BlockSpec((_BLK, _NE), lambda i: (i, 0)),
    ],
    out_specs=pl.BlockSpec((1, 1, _BLK), lambda i: (i, 0, 0)),
    out_shape=jax.ShapeDtypeStruct((_NB, 1, _BLK), jnp.float32),
)


def kernel(t_users, user_id, t_items, item_id, F_u, E_u, Tw_u, Tb_u,
           F_i, E_i, Tw_i, Tb_i, final_bias):
    uid2 = user_id.astype(jnp.int32).reshape(_B // _CH, _CH)
    iid2 = item_id.astype(jnp.int32).reshape(_B // _CH, _CH)
    e_u, e_i = _sc_gather(uid2, iid2, E_u, E_i)
    c0 = (1.0 + F_u[0, 0] * F_i[0, 0] + final_bias[0]).reshape(1)
    out = _tc_combine(c0, Tb_u, Tb_i, Tw_u, Tw_i, t_users, t_items, e_u, e_i)
    return out.reshape(_B)


# speedup vs baseline: 2.5270x; 1.0326x over previous
import functools

import jax
import jax.numpy as jnp
from jax import lax
from jax.experimental import pallas as pl
from jax.experimental.pallas import tpu as pltpu
from jax.experimental.pallas import tpu_sc as plsc

_B = 16384
_FV = 64
_NE = 16
_L = 16
_NC = 2
_NS = 16
_NW = _NC * _NS
_BPW = _B // _NW
_CH = 128
_NCH = _BPW // _CH
_NG = _BPW // _L
_PAD = _L + 1


def _sc_body(uid_hbm, iid_hbm, eu_hbm, ei_hbm, t_u_hbm, t_i_hbm,
             twu_hbm, twi_hbm, cst_hbm, out_hbm,
             uidx_v, iidx_v, eur_v, eir_v, tu_v, ti_v,
             scu_v, sci_v, sce_v, out_v, twu_v, twi_v, cst_v, sem):
    wid = lax.axis_index("s") * _NC + lax.axis_index("c")
    base = wid * _BPW
    crow = wid * _NCH
    pltpu.sync_copy(uid_hbm.at[pl.ds(crow, _NCH)], uidx_v)
    pltpu.sync_copy(iid_hbm.at[pl.ds(crow, _NCH)], iidx_v)
    pltpu.sync_copy(twu_hbm, twu_v)
    pltpu.sync_copy(twi_hbm, twi_v)
    pltpu.sync_copy(cst_hbm, cst_v)
    cps = [
        pltpu.async_copy(t_u_hbm.at[pl.ds(base, _BPW)], tu_v, sem),
        pltpu.async_copy(t_i_hbm.at[pl.ds(base, _BPW)], ti_v, sem),
    ]
    for j in range(_NCH):
        cps.append(pltpu.async_copy(
            eu_hbm.at[uidx_v.at[j]], eur_v.at[pl.ds(j * _CH, _CH)], sem))
        cps.append(pltpu.async_copy(
            ei_hbm.at[iidx_v.at[j]], eir_v.at[pl.ds(j * _CH, _CH)], sem))
    for cp in cps:
        cp.wait()

    wu = [twu_v[c, :] for c in range(_FV // _L)]
    wi = [twi_v[c, :] for c in range(_FV // _L)]

    @plsc.parallel_loop(0, _BPW, 1, unroll=4)
    def _rows(r):
        su = tu_v[r, pl.ds(0, _L)] * wu[0]
        si = ti_v[r, pl.ds(0, _L)] * wi[0]
        for c in range(1, _FV // _L):
            su = su + tu_v[r, pl.ds(c * _L, _L)] * wu[c]
            si = si + ti_v[r, pl.ds(c * _L, _L)] * wi[c]
        scu_v[r, pl.ds(0, _L)] = su
        sci_v[r, pl.ds(0, _L)] = si
        sce_v[r, pl.ds(0, _L)] = eur_v[r, :] * eir_v[r, :]

    lanes = lax.iota(jnp.int32, _L)
    c0 = cst_v[0, :]
    tbu = cst_v[1, :]
    tbi = cst_v[2, :]

    @plsc.parallel_loop(0, _NG, 1, unroll=2)
    def _groups(g):
        rows = g * _L + lanes
        cols0 = jnp.zeros((_L,), jnp.int32)
        au = plsc.load_gather(scu_v, [rows, cols0])
        ai = plsc.load_gather(sci_v, [rows, cols0])
        ae = plsc.load_gather(sce_v, [rows, cols0])
        for c in range(1, _L):
            colsc = jnp.full((_L,), c, jnp.int32)
            au = au + plsc.load_gather(scu_v, [rows, colsc])
            ai = ai + plsc.load_gather(sci_v, [rows, colsc])
            ae = ae + plsc.load_gather(sce_v, [rows, colsc])
        out_v[pl.ds(g * _L, _L)] = (au + tbu) * (ai + tbi) + ae + c0

    pltpu.sync_copy(out_v, out_hbm.at[pl.ds(base, _BPW)])


_sc_kernel = functools.partial(
    pl.kernel,
    mesh=plsc.VectorSubcoreMesh(core_axis_name="c", subcore_axis_name="s"),
    out_type=jax.ShapeDtypeStruct((_B,), jnp.float32),
    scratch_types=[
        pltpu.VMEM((_NCH, _CH), jnp.int32),
        pltpu.VMEM((_NCH, _CH), jnp.int32),
        pltpu.VMEM((_BPW, _NE), jnp.float32),
        pltpu.VMEM((_BPW, _NE), jnp.float32),
        pltpu.VMEM((_BPW, _FV), jnp.float32),
        pltpu.VMEM((_BPW, _FV), jnp.float32),
        pltpu.VMEM((_BPW, _PAD), jnp.float32),
        pltpu.VMEM((_BPW, _PAD), jnp.float32),
        pltpu.VMEM((_BPW, _PAD), jnp.float32),
        pltpu.VMEM((_BPW,), jnp.float32),
        pltpu.VMEM((_FV // _L, _L), jnp.float32),
        pltpu.VMEM((_FV // _L, _L), jnp.float32),
        pltpu.VMEM((4, _L), jnp.float32),
        pltpu.SemaphoreType.DMA,
    ],
    compiler_params=pltpu.CompilerParams(use_tc_tiling_on_sc=False,
                                         needs_layout_passes=False),
)(_sc_body)


def kernel(t_users, user_id, t_items, item_id, F_u, E_u, Tw_u, Tb_u,
           F_i, E_i, Tw_i, Tb_i, final_bias):
    uid2 = user_id.astype(jnp.int32).reshape(_B // _CH, _CH)
    iid2 = item_id.astype(jnp.int32).reshape(_B // _CH, _CH)
    c0 = 1.0 + F_u[0, 0] * F_i[0, 0] + final_bias[0]
    cst = jnp.stack([
        jnp.broadcast_to(c0, (_L,)),
        jnp.broadcast_to(Tb_u[0], (_L,)),
        jnp.broadcast_to(Tb_i[0], (_L,)),
        jnp.zeros((_L,), jnp.float32),
    ])
    return _sc_kernel(uid2, iid2, E_u, E_i, t_users, t_items,
                      Tw_u.reshape(_FV // _L, _L), Tw_i.reshape(_FV // _L, _L),
                      cst)


# speedup vs baseline: 4.3306x; 1.7137x over previous
import functools

import jax
import jax.numpy as jnp
from jax import lax
from jax.experimental import pallas as pl
from jax.experimental.pallas import tpu as pltpu
from jax.experimental.pallas import tpu_sc as plsc

_B = 16384
_FV = 64
_NE = 16
_NV = 1001
_L = 16
_NC = 2
_NS = 16
_NW = _NC * _NS
_BPW = _B // _NW
_CH = 128
_NCH = _BPW // _CH
_NG = _BPW // _L

_BLK = 2048
_NB = _B // _BLK


def _sc_body(uid_hbm, iid_hbm, euT_hbm, eiT_hbm, out_hbm,
             uidx_v, iidx_v, eu_v, ei_v, out_v, sem):
    wid = lax.axis_index("s") * _NC + lax.axis_index("c")
    base = wid * _BPW
    crow = wid * _NCH
    pltpu.sync_copy(uid_hbm.at[pl.ds(crow, _NCH)], uidx_v)
    pltpu.sync_copy(iid_hbm.at[pl.ds(crow, _NCH)], iidx_v)
    cps = [pltpu.async_copy(euT_hbm, eu_v, sem),
           pltpu.async_copy(eiT_hbm, ei_v, sem)]
    for cp in cps:
        cp.wait()
    for g in range(_NG):
        uu = uidx_v[g // 8, pl.ds((g % 8) * _L, _L)]
        vv = iidx_v[g // 8, pl.ds((g % 8) * _L, _L)]
        acc = None
        for k in range(_NE):
            kk = jnp.full((_L,), k, jnp.int32)
            p = plsc.load_gather(eu_v, [kk, uu]) * plsc.load_gather(ei_v, [kk, vv])
            acc = p if acc is None else acc + p
        out_v[pl.ds(g * _L, _L)] = acc
    pltpu.sync_copy(out_v, out_hbm.at[pl.ds(base, _BPW)])


_sc_edot = functools.partial(
    pl.kernel,
    mesh=plsc.VectorSubcoreMesh(core_axis_name="c", subcore_axis_name="s"),
    out_type=jax.ShapeDtypeStruct((_B,), jnp.float32),
    scratch_types=[
        pltpu.VMEM((_NCH, _CH), jnp.int32),
        pltpu.VMEM((_NCH, _CH), jnp.int32),
        pltpu.VMEM((_NE, _NV), jnp.float32),
        pltpu.VMEM((_NE, _NV), jnp.float32),
        pltpu.VMEM((_BPW,), jnp.float32),
        pltpu.SemaphoreType.DMA,
    ],
    compiler_params=pltpu.CompilerParams(use_tc_tiling_on_sc=False,
                                         needs_layout_passes=False),
)(_sc_body)


def _tc_body(c0_ref, tbu_ref, tbi_ref, twu_ref, twi_ref,
             tu_ref, ti_ref, ed_ref, out_ref):
    su = jnp.sum(tu_ref[...] * twu_ref[...], axis=0) + tbu_ref[0]
    si = jnp.sum(ti_ref[...] * twi_ref[...], axis=0) + tbi_ref[0]
    out_ref[...] = su * si + ed_ref[...] + c0_ref[0]


_tc_combine = pl.pallas_call(
    _tc_body,
    grid=(_NB,),
    in_specs=[
        pl.BlockSpec(memory_space=pltpu.SMEM),
        pl.BlockSpec(memory_space=pltpu.SMEM),
        pl.BlockSpec(memory_space=pltpu.SMEM),
        pl.BlockSpec((_FV, 1), lambda i: (0, 0)),
        pl.BlockSpec((_FV, 1), lambda i: (0, 0)),
        pl.BlockSpec((_FV, _BLK), lambda i: (0, i)),
        pl.BlockSpec((_FV, _BLK), lambda i: (0, i)),
        pl.BlockSpec((_BLK,), lambda i: (i,)),
    ],
    out_specs=pl.BlockSpec((_BLK,), lambda i: (i,)),
    out_shape=jax.ShapeDtypeStruct((_B,), jnp.float32),
)


def kernel(t_users, user_id, t_items, item_id, F_u, E_u, Tw_u, Tb_u,
           F_i, E_i, Tw_i, Tb_i, final_bias):
    uid2 = user_id.astype(jnp.int32).reshape(_B // _CH, _CH)
    iid2 = item_id.astype(jnp.int32).reshape(_B // _CH, _CH)
    e_dot = _sc_edot(uid2, iid2, E_u.T, E_i.T)
    c0 = (1.0 + F_u[0, 0] * F_i[0, 0] + final_bias[0]).reshape(1)
    return _tc_combine(c0, Tb_u, Tb_i, Tw_u.reshape(_FV, 1), Tw_i.reshape(_FV, 1),
                       t_users.T, t_items.T, e_dot)
